# Initial kernel scaffold; baseline (speedup 1.0000x reference)
import jax
import jax.numpy as jnp
from jax.experimental import pallas as pl


def kernel(x_list, edge_index_list, batch_list, W1, b1, W2, b2, Wih0, Whh0, bih0, bhh0, Wih1, Whh1, bih1, bhh1, Wlin, blin):
    raise NotImplementedError("write your pallas kernel here")


# speedup vs baseline: 1.0009x; 1.0009x over previous
import functools

import jax
import jax.numpy as jnp
from jax.experimental import pallas as pl
from jax.experimental.pallas import tpu as pltpu

T = 8
N = 10000
E = 320000
D = 128
H = 128
HL = 128
C = 10
G = 16


def _sigmoid(x):
    return 1.0 / (1.0 + jnp.exp(-x))


def _lstm_head_body(seq_ref, wih0, whh0, b0, wih1, whh1, b1, wlin, blin, out_ref):
    def cell(x, h, c, wih, whh, b):
        g = jnp.dot(x, wih[...], preferred_element_type=jnp.float32)
        g = g + jnp.dot(h, whh[...], preferred_element_type=jnp.float32) + b[...]
        i = _sigmoid(g[:, 0:HL])
        f = _sigmoid(g[:, HL:2 * HL])
        gg = jnp.tanh(g[:, 2 * HL:3 * HL])
        o = _sigmoid(g[:, 3 * HL:4 * HL])
        c_new = f * c + i * gg
        h_new = o * jnp.tanh(c_new)
        return h_new, c_new

    def step(t, carry):
        h0, c0, h1, c1 = carry
        x = seq_ref[pl.ds(t, 1), :]
        h0, c0 = cell(x, h0, c0, wih0, whh0, b0)
        h1, c1 = cell(h0, h1, c1, wih1, whh1, b1)
        return (h0, c0, h1, c1)

    z = jnp.zeros((1, HL), dtype=jnp.float32)
    h0, c0, h1, c1 = jax.lax.fori_loop(0, T * G, step, (z, z, z, z))
    res = jnp.dot(h1, wlin[...], preferred_element_type=jnp.float32) + blin[...]
    out_ref[...] = jnp.broadcast_to(res, (8, 128))


def _lstm_head(seq, Wih0, Whh0, bih0, bhh0, Wih1, Whh1, bih1, bhh1, Wlin, blin):
    b0 = (bih0 + bhh0)[None, :]
    b1 = (bih1 + bhh1)[None, :]
    wlin_pad = jnp.zeros((HL, 128), jnp.float32).at[:, :C].set(Wlin.T)
    blin_pad = jnp.zeros((1, 128), jnp.float32).at[:, :C].set(blin[None, :])
    out = pl.pallas_call(
        _lstm_head_body,
        out_shape=jax.ShapeDtypeStruct((8, 128), jnp.float32),
    )(seq, Wih0.T, Whh0.T, b0, Wih1.T, Whh1.T, b1, wlin_pad, blin_pad)
    return out[0:1, 0:C]


def _gcn_conv(x, edge_index, W, b):
    num_nodes = x.shape[0]
    loop = jnp.arange(num_nodes)
    src = jnp.concatenate([edge_index[0], loop])
    dst = jnp.concatenate([edge_index[1], loop])
    deg = jnp.zeros((num_nodes,), dtype=x.dtype).at[dst].add(1.0)
    dinv = jax.lax.rsqrt(deg)
    norm = dinv[src] * dinv[dst]
    xw = x @ W
    msg = jnp.take(xw, src, axis=0) * norm[:, None]
    out = jnp.zeros((num_nodes, W.shape[1]), dtype=x.dtype).at[dst].add(msg)
    return out + b


def kernel(x_list, edge_index_list, batch_list, W1, b1, W2, b2, Wih0, Whh0,
           bih0, bhh0, Wih1, Whh1, bih1, bhh1, Wlin, blin):
    embs = []
    for i in range(T):
        x = jax.nn.relu(_gcn_conv(x_list[i], edge_index_list[i], W1, b1))
        x = jax.nn.relu(_gcn_conv(x, edge_index_list[i], W2, b2))
        sums = jax.ops.segment_sum(x, batch_list[i], num_segments=G)
        counts = jax.ops.segment_sum(jnp.ones((N,), jnp.float32), batch_list[i], num_segments=G)
        embs.append(sums / jnp.maximum(counts, 1.0)[:, None])
    seq = jnp.concatenate(embs, axis=0)
    return _lstm_head(seq, Wih0, Whh0, bih0, bhh0, Wih1, Whh1, bih1, bhh1, Wlin, blin)


# speedup vs baseline: 1.5666x; 1.5652x over previous
import functools

import jax
import jax.numpy as jnp
from jax import lax
from jax.experimental import pallas as pl
from jax.experimental.pallas import tpu as pltpu
from jax.experimental.pallas import tpu_sc as plsc

T = 8
N = 10000
E = 320000
D = 128
H = 128
HL = 128
C = 10
G = 16

HALF = 5120
APAD = 5376
JUNK = 5120
NPAD = 10240
EP = 327680
CCONV = 160
CDEG = 80
NB = 2

_f32 = jnp.float32
_i32 = jnp.int32

_mesh = plsc.VectorSubcoreMesh(core_axis_name="c", subcore_axis_name="s")


_EW = EP // 32


_HR = NPAD * 8 // 128


@functools.partial(
    pl.kernel,
    out_type=jax.ShapeDtypeStruct((2, 16, T, _HR, 128), _f32),
    mesh=_mesh,
    scratch_types=[
        pltpu.VMEM((_HR, 128), _f32),
        pltpu.VMEM((_EW,), _i32),
        pltpu.VMEM((_EW,), _i32),
    ],
    compiler_params=pltpu.CompilerParams(needs_layout_passes=False),
)
def _deg_sc(idxr_hbm, idxc_hbm, zeros_hbm, out_hbm, hist_v, idxr_v, idxc_v):
    cid = lax.axis_index("c")
    sid = lax.axis_index("s")
    wid = cid * 16 + sid
    ones = jnp.ones((16,), _f32)
    for t in range(T):
        pltpu.sync_copy(zeros_hbm, hist_v)
        pltpu.sync_copy(idxr_hbm.at[t, wid], idxr_v)
        pltpu.sync_copy(idxc_hbm.at[t, wid], idxc_v)

        def it(i, carry):
            ids_r = idxr_v[pl.ds(i * 16, 16)]
            ids_c = idxc_v[pl.ds(i * 16, 16)]
            plsc.addupdate_scatter(hist_v, [ids_r, ids_c], ones)
            return carry

        lax.fori_loop(0, _EW // 16, it, 0)
        pltpu.sync_copy(hist_v, out_hbm.at[cid, sid, t])


@functools.partial(
    pl.kernel,
    out_type=jax.ShapeDtypeStruct((T * N, 128), _f32),
    mesh=_mesh,
    scratch_types=[
        pltpu.VMEM((CCONV, 128), _i32),
        pltpu.VMEM((CCONV, 128), _i32),
        pltpu.VMEM((NB, 128, 128), _f32),
        pltpu.VMEM((80, 128), _f32),
        pltpu.VMEM_SHARED((APAD, 128), _f32),
        pltpu.SemaphoreType.DMA,
    ],
)
def _conv_sc(y_hbm, srcc_hbm, dstc_hbm, out_hbm, src_v, dst_v, bufs_v, stage_v, acc_sh, gsem):
    cid = lax.axis_index("c")
    sid = lax.axis_index("s")
    last1 = jnp.logical_and(cid == 1, sid == 15)
    for t in range(T):
        base = t * N + cid * HALF + sid * 320

        @pl.when(jnp.logical_not(last1))
        def _():
            pltpu.sync_copy(y_hbm.at[pl.ds(base, 320)], acc_sh.at[pl.ds(sid * 320, 320)])

        @pl.when(last1)
        def _():
            pltpu.sync_copy(y_hbm.at[pl.ds(base, 80)], acc_sh.at[pl.ds(15 * 320, 80)])

        plsc.subcore_barrier()
        pltpu.sync_copy(srcc_hbm.at[t, sid], src_v)
        pltpu.sync_copy(dstc_hbm.at[t, cid, sid], dst_v)
        for b in range(NB):
            pltpu.async_copy(y_hbm.at[src_v.at[b]], bufs_v.at[b], gsem)

        def group(gi, carry):
            for b in range(NB):
                j = gi * NB + b
                pltpu.make_async_copy(y_hbm.at[src_v.at[j]], bufs_v.at[b], gsem).wait()
                pltpu.sync_copy(bufs_v.at[b], acc_sh.at[dst_v.at[j]], add=True)
                nj = j + NB

                @pl.when(nj < CCONV)
                def _():
                    pltpu.async_copy(y_hbm.at[src_v.at[nj]], bufs_v.at[b], gsem)
            return carry

        lax.fori_loop(0, CCONV // NB, group, 0)
        plsc.subcore_barrier()

        @pl.when(jnp.logical_not(last1))
        def _():
            for hh in range(4):
                pltpu.sync_copy(acc_sh.at[pl.ds(sid * 320 + hh * 80, 80)], stage_v)
                pltpu.sync_copy(stage_v, out_hbm.at[pl.ds(base + hh * 80, 80)])

        @pl.when(last1)
        def _():
            pltpu.sync_copy(acc_sh.at[pl.ds(15 * 320, 80)], stage_v)
            pltpu.sync_copy(stage_v, out_hbm.at[pl.ds(base, 80)])


_BLK = 640
_NBLK = (T * N) // _BLK


def _tc_xw_body(x_ref, w_ref, o_ref):
    o_ref[...] = jnp.dot(x_ref[...], w_ref[...], preferred_element_type=_f32)


def _tc_xw(x, w):
    return pl.pallas_call(
        _tc_xw_body,
        grid=(_NBLK,),
        in_specs=[
            pl.BlockSpec((_BLK, 128), lambda i: (i, 0)),
            pl.BlockSpec((128, 128), lambda i: (0, 0)),
        ],
        out_specs=pl.BlockSpec((_BLK, 128), lambda i: (i, 0)),
        out_shape=jax.ShapeDtypeStruct((T * N, 128), _f32),
    )(x, w)


def _tc_scale_body(y_ref, degp_ref, yp_ref, dinv_ref):
    s = degp_ref[0, 0]
    for w in range(1, 32):
        s = s + degp_ref[w, 0]
    deg = jnp.sum(s, axis=1, keepdims=True) + 1.0
    dinv = lax.rsqrt(deg)
    yp_ref[...] = y_ref[...] * dinv
    dinv_ref[...] = jnp.broadcast_to(dinv, (400, 16))


def _tc_scale(y1, degp):
    return pl.pallas_call(
        _tc_scale_body,
        grid=(T, 25),
        in_specs=[
            pl.BlockSpec((400, 128), lambda t, i: (t * 25 + i, 0)),
            pl.BlockSpec((32, 1, 400, 8), lambda t, i: (0, t, i, 0)),
        ],
        out_specs=[
            pl.BlockSpec((400, 128), lambda t, i: (t * 25 + i, 0)),
            pl.BlockSpec((400, 16), lambda t, i: (t * 25 + i, 0)),
        ],
        out_shape=[
            jax.ShapeDtypeStruct((T * N, 128), _f32),
            jax.ShapeDtypeStruct((T * N, 16), _f32),
        ],
    )(y1, degp)


def _tc_mid_body(acc_ref, dinv_ref, b_ref, w_ref, yp_ref):
    dinv = dinv_ref[:, 0:1]
    h = acc_ref[...] * dinv + b_ref[...]
    h = jnp.maximum(h, 0.0)
    yp_ref[...] = jnp.dot(h, w_ref[...], preferred_element_type=_f32) * dinv


def _tc_mid(acc1, dinv16, b1, W2):
    return pl.pallas_call(
        _tc_mid_body,
        grid=(_NBLK,),
        in_specs=[
            pl.BlockSpec((_BLK, 128), lambda i: (i, 0)),
            pl.BlockSpec((_BLK, 16), lambda i: (i, 0)),
            pl.BlockSpec((1, 128), lambda i: (0, 0)),
            pl.BlockSpec((128, 128), lambda i: (0, 0)),
        ],
        out_specs=pl.BlockSpec((_BLK, 128), lambda i: (i, 0)),
        out_shape=jax.ShapeDtypeStruct((T * N, 128), _f32),
    )(acc1, dinv16, b1, W2)


def _sigmoid(x):
    return 1.0 / (1.0 + jnp.exp(-x))


def _tc_tail_body(acc_ref, dinv_ref, b_ref, batch_ref, wih0, whh0, b0, wih1,
                  whh1, b1l, wlin, blin, out_ref, seq_s, y0_s, xg_s):
    t = pl.program_id(0)
    h2 = acc_ref[...] * dinv_ref[:, 0:1] + b_ref[...]
    h2 = jnp.maximum(h2, 0.0)
    bt = jnp.broadcast_to(batch_ref[0], (G, N))
    gid = lax.broadcasted_iota(_i32, (G, 1), 0)
    P = (bt == gid).astype(_f32)
    sums = jnp.dot(P, h2, preferred_element_type=_f32)
    counts = jnp.sum(P, axis=1, keepdims=True)
    seq_s[pl.ds(t * G, G), :] = sums / jnp.maximum(counts, 1.0)

    @pl.when(t == T - 1)
    def _():
        def cell(gates, h, c, whh):
            g = gates + jnp.dot(h, whh[...], preferred_element_type=_f32)
            i = _sigmoid(g[:, 0:HL])
            f = _sigmoid(g[:, HL:2 * HL])
            gg = jnp.tanh(g[:, 2 * HL:3 * HL])
            o = _sigmoid(g[:, 3 * HL:4 * HL])
            c_new = f * c + i * gg
            return o * jnp.tanh(c_new), c_new

        z = jnp.zeros((1, HL), _f32)
        xg_s[...] = jnp.dot(seq_s[...], wih0[...], preferred_element_type=_f32) + b0[...]

        def step0(i, carry):
            h, c = carry
            h, c = cell(xg_s[pl.ds(i, 1), :], h, c, whh0)
            y0_s[pl.ds(i, 1), :] = h
            return (h, c)

        lax.fori_loop(0, T * G, step0, (z, z))
        xg_s[...] = jnp.dot(y0_s[...], wih1[...], preferred_element_type=_f32) + b1l[...]

        def step1(i, carry):
            h, c = carry
            return cell(xg_s[pl.ds(i, 1), :], h, c, whh1)

        hf, _cf = lax.fori_loop(0, T * G, step1, (z, z))
        res = jnp.dot(hf, wlin[...], preferred_element_type=_f32) + blin[...]
        out_ref[...] = jnp.broadcast_to(res, (8, 128))


def _tc_tail(acc2, dinv16, b2, batch3, wih0T, whh0T, b0, wih1T, whh1T, b1l, wlin_pad, blin_pad):
    full = lambda shape: pl.BlockSpec(shape, lambda t: tuple(0 for _ in shape))
    return pl.pallas_call(
        _tc_tail_body,
        grid=(T,),
        in_specs=[
            pl.BlockSpec((N, 128), lambda t: (t, 0)),
            pl.BlockSpec((N, 16), lambda t: (t, 0)),
            full((1, 128)),
            pl.BlockSpec((1, 1, N), lambda t: (t, 0, 0)),
            full((128, 4 * HL)),
            full((HL, 4 * HL)),
            full((1, 4 * HL)),
            full((HL, 4 * HL)),
            full((HL, 4 * HL)),
            full((1, 4 * HL)),
            full((HL, 128)),
            full((1, 128)),
        ],
        out_specs=pl.BlockSpec((8, 128), lambda t: (0, 0)),
        out_shape=jax.ShapeDtypeStruct((8, 128), _f32),
        scratch_shapes=[
            pltpu.VMEM((T * G, H), _f32),
            pltpu.VMEM((T * G, HL), _f32),
            pltpu.VMEM((T * G, 4 * HL), _f32),
        ],
    )(acc2, dinv16, b2, batch3, wih0T, whh0T, b0, wih1T, whh1T, b1l, wlin_pad, blin_pad)


def kernel(x_list, edge_index_list, batch_list, W1, b1, W2, b2, Wih0, Whh0,
           bih0, bhh0, Wih1, Whh1, bih1, bhh1, Wlin, blin):
    ei = edge_index_list.astype(_i32)
    pad = EP - E
    src = jnp.concatenate([ei[:, 0, :], jnp.zeros((T, pad), _i32)], axis=1)
    dst = jnp.concatenate([ei[:, 1, :], jnp.full((T, pad), N, _i32)], axis=1)
    src_g = src + (jnp.arange(T, dtype=_i32) * N)[:, None]
    src_conv = src_g.reshape(T, 16, CCONV, 128)
    dst_c0 = jnp.where(dst < HALF, dst, JUNK)
    dst_c1 = jnp.where(dst >= HALF, dst - HALF, JUNK)
    dst_conv = jnp.stack([dst_c0, dst_c1], axis=1).reshape(T, 2, 16, CCONV, 128)
    dst_deg = dst.reshape(T, 32, CDEG, 128)
    ones16 = jnp.ones((128, 16), _f32)
    zeros16 = jnp.zeros((128, 16), _f32)
    iota_np = jnp.arange(NPAD, dtype=_i32).reshape(NPAD // 128, 128)
    batch3 = batch_list.astype(_i32).reshape(T, 1, N)

    xflat = x_list.reshape(T * N, D)
    y1 = _tc_xw(xflat, W1)
    idx8 = dst * 8 + (jnp.arange(EP, dtype=_i32) % 8)[None, :]
    idxr = (idx8 >> 7).reshape(T, 32, _EW)
    idxc = (idx8 & 127).reshape(T, 32, _EW)
    zerosh = jnp.zeros((_HR, 128), _f32)
    degp = _deg_sc(idxr, idxc, zerosh).reshape(32, T, NPAD, 8)
    yp1, dinv16 = _tc_scale(y1, degp)
    tN = (jnp.arange(T, dtype=_i32) * N)[:, None]
    flat_dst = jnp.where(dst < N, dst + tN, T * N).reshape(-1)
    flat_src = src_g.reshape(-1)

    def _conv_jnp(yp):
        return yp.at[flat_dst].add(jnp.take(yp, flat_src, axis=0), mode="drop")

    acc1 = _conv_jnp(yp1)
    yp2 = _tc_mid(acc1, dinv16, b1.reshape(1, 128), W2)
    acc2 = _conv_jnp(yp2)

    b0 = (bih0 + bhh0).reshape(1, 4 * HL)
    b1l = (bih1 + bhh1).reshape(1, 4 * HL)
    wlin_pad = jnp.zeros((HL, 128), _f32).at[:, :C].set(Wlin.T)
    blin_pad = jnp.zeros((1, 128), _f32).at[:, :C].set(blin[None, :])
    out8 = _tc_tail(acc2, dinv16, b2.reshape(1, 128), batch3, Wih0.T, Whh0.T,
                    b0, Wih1.T, Whh1.T, b1l, wlin_pad, blin_pad)
    return out8[0:1, 0:C]
